# baseline (speedup 1.0000x reference)
import jax
import jax.numpy as jnp
from jax.experimental import pallas as pl
from jax.experimental.pallas import tpu as pltpu

S = 64
_NEG = -1e30


def _body(x_ref, seg_ref, w1_ref, b1_ref, w2_ref, out_ref, m_ref, den_ref):
    i = pl.program_id(0)
    nb = pl.num_programs(0)
    x = x_ref[...]
    b = x.shape[0]

    y = jax.lax.dot(x, w1_ref[...], preferred_element_type=jnp.float32)
    z = jnp.maximum(y, b1_ref[...])
    g = jnp.sum(z * w2_ref[...], axis=1)

    seg = seg_ref[0, 0, :]
    onehot = seg[:, None] == jax.lax.broadcasted_iota(jnp.int32, (b, S), 1)
    gmat = jnp.where(onehot, g[:, None], _NEG)
    bmax = jnp.max(gmat, axis=0)

    @pl.when(i == 0)
    def _init():
        m_ref[0, :] = jnp.full((S,), _NEG, jnp.float32)
        den_ref[0, :] = jnp.zeros((S,), jnp.float32)
        out_ref[...] = jnp.zeros_like(out_ref)

    m_old = m_ref[0, :]
    m_new = jnp.maximum(m_old, bmax)
    scale = jnp.exp(m_old - m_new)
    m_ref[0, :] = m_new

    m_sub = jnp.where(m_new == _NEG, 0.0, m_new)
    p = jnp.exp(gmat - m_sub[None, :])

    den_ref[0, :] = den_ref[0, :] * scale + jnp.sum(p, axis=0)
    out_ref[...] = out_ref[...] * scale[:, None] + jax.lax.dot_general(
        p, x, (((0,), (0,)), ((), ())),
        preferred_element_type=jnp.float32)

    @pl.when(i == nb - 1)
    def _finish():
        den = den_ref[0, :]
        out_ref[...] = jnp.where(den[:, None] > 0.0,
                                 out_ref[...] / den[:, None], 0.0)


def kernel(x, batch, W1, b1, W2, b2):
    n, d = x.shape
    blk = max(v for v in range(8, min(n, 4096) + 1, 8) if n % v == 0)
    nb = n // blk
    seg3 = batch.astype(jnp.int32).reshape(nb, 1, blk)
    return pl.pallas_call(
        _body,
        grid=(nb,),
        in_specs=[
            pl.BlockSpec((blk, d), lambda i: (i, 0)),
            pl.BlockSpec((1, 1, blk), lambda i: (i, 0, 0)),
            pl.BlockSpec((d, d), lambda i: (0, 0)),
            pl.BlockSpec((1, d), lambda i: (0, 0)),
            pl.BlockSpec((1, d), lambda i: (0, 0)),
        ],
        out_specs=pl.BlockSpec((S, d), lambda i: (0, 0)),
        out_shape=jax.ShapeDtypeStruct((S, d), jnp.float32),
        scratch_shapes=[
            pltpu.VMEM((1, S), jnp.float32),
            pltpu.VMEM((1, S), jnp.float32),
        ],
        compiler_params=pltpu.CompilerParams(
            dimension_semantics=("arbitrary",)),
    )(x, seg3, W1, (-b1).reshape(1, d), W2.reshape(1, d))


# speedup vs baseline: 1.0278x; 1.0278x over previous
import jax
import jax.numpy as jnp
from jax.experimental import pallas as pl
from jax.experimental.pallas import tpu as pltpu

S = 64
_NEG = -1e30


def _body(x_ref, seg_ref, w1_ref, b1_ref, w2_ref, out_ref, m_ref, den_ref):
    i = pl.program_id(0)
    nb = pl.num_programs(0)
    x = x_ref[...]
    b = x.shape[0]

    y = jax.lax.dot(x, w1_ref[...], preferred_element_type=jnp.float32)
    z = jnp.maximum(y, b1_ref[...])
    g = jnp.sum(z * w2_ref[...], axis=1)

    seg = seg_ref[0, 0, :]
    onehot = seg[:, None] == jax.lax.broadcasted_iota(jnp.int32, (b, S), 1)
    gmat = jnp.where(onehot, g[:, None], _NEG)
    bmax = jnp.max(gmat, axis=0)

    @pl.when(i == 0)
    def _init():
        m_ref[0, :] = jnp.full((S,), _NEG, jnp.float32)
        den_ref[0, :] = jnp.zeros((S,), jnp.float32)
        out_ref[...] = jnp.zeros_like(out_ref)

    m_old = m_ref[0, :]
    m_new = jnp.maximum(m_old, bmax)
    scale = jnp.exp(m_old - m_new)
    m_ref[0, :] = m_new

    m_sub = jnp.where(m_new == _NEG, 0.0, m_new)
    p = jnp.exp(gmat - m_sub[None, :])

    den_ref[0, :] = den_ref[0, :] * scale + jnp.sum(p, axis=0)
    out_ref[...] = out_ref[...] * scale[:, None] + jax.lax.dot_general(
        p, x, (((0,), (0,)), ((), ())),
        preferred_element_type=jnp.float32)

    @pl.when(i == nb - 1)
    def _finish():
        den = den_ref[0, :]
        out_ref[...] = jnp.where(den[:, None] > 0.0,
                                 out_ref[...] / den[:, None], 0.0)


def kernel(x, batch, W1, b1, W2, b2):
    n, d = x.shape
    blk = max(v for v in range(8, min(n, 5120) + 1, 8) if n % v == 0)
    nb = n // blk
    seg3 = batch.astype(jnp.int32).reshape(nb, 1, blk)
    return pl.pallas_call(
        _body,
        grid=(nb,),
        in_specs=[
            pl.BlockSpec((blk, d), lambda i: (i, 0)),
            pl.BlockSpec((1, 1, blk), lambda i: (i, 0, 0)),
            pl.BlockSpec((d, d), lambda i: (0, 0)),
            pl.BlockSpec((1, d), lambda i: (0, 0)),
            pl.BlockSpec((1, d), lambda i: (0, 0)),
        ],
        out_specs=pl.BlockSpec((S, d), lambda i: (0, 0)),
        out_shape=jax.ShapeDtypeStruct((S, d), jnp.float32),
        scratch_shapes=[
            pltpu.VMEM((1, S), jnp.float32),
            pltpu.VMEM((1, S), jnp.float32),
        ],
        compiler_params=pltpu.CompilerParams(
            dimension_semantics=("arbitrary",)),
    )(x, seg3, W1, (-b1).reshape(1, d), W2.reshape(1, d))
